# baseline (speedup 1.0000x reference)
import jax
import jax.numpy as jnp
from jax.experimental import pallas as pl
from jax.experimental.pallas import tpu as pltpu

_BLOCK_ROWS = 256


def _fused_kernel(x1_ref, x2_ref, w_ref, l1a_ref, l1b_ref, l2a_ref, l2b_ref,
                  b_ref, o1_ref, o2_ref, s1_ref, s2_ref):
    @pl.when(pl.program_id(0) == 0)
    def _init():
        w = w_ref[...]
        s1_ref[...] = jnp.dot(x1_ref[...], w, preferred_element_type=jnp.float32)
        s2_ref[...] = jnp.dot(x2_ref[...], w, preferred_element_type=jnp.float32)

    kh = s1_ref.shape[0] // 2
    b = b_ref[...]
    o1_ref[...] = (jnp.dot(l1a_ref[...], s1_ref[:kh, :],
                           preferred_element_type=jnp.float32)
                   + jnp.dot(l1b_ref[...], s1_ref[kh:, :],
                             preferred_element_type=jnp.float32) + b)
    o2_ref[...] = (jnp.dot(l2a_ref[...], s2_ref[:kh, :],
                           preferred_element_type=jnp.float32)
                   + jnp.dot(l2b_ref[...], s2_ref[kh:, :],
                             preferred_element_type=jnp.float32) + b)


def kernel(node_input, hyperedge_input, node_lap, hyperedge_lap, weight, bias):
    n, f_in = node_input.shape
    m = hyperedge_input.shape[0]
    f_out = weight.shape[1]

    bias2d = bias.reshape(1, f_out)
    blk = _BLOCK_ROWS
    half = n // 2
    o1, o2 = pl.pallas_call(
        _fused_kernel,
        grid=(n // blk,),
        in_specs=[
            pl.BlockSpec((n, f_in), lambda i: (0, 0)),
            pl.BlockSpec((m, f_in), lambda i: (0, 0)),
            pl.BlockSpec((f_in, f_out), lambda i: (0, 0)),
            pl.BlockSpec((blk, half), lambda i: (i, 0)),
            pl.BlockSpec((blk, half), lambda i: (i, 1)),
            pl.BlockSpec((blk, half), lambda i: (i, 0)),
            pl.BlockSpec((blk, half), lambda i: (i, 1)),
            pl.BlockSpec((1, f_out), lambda i: (0, 0)),
        ],
        out_specs=(
            pl.BlockSpec((blk, f_out), lambda i: (i, 0)),
            pl.BlockSpec((blk, f_out), lambda i: (i, 0)),
        ),
        out_shape=(
            jax.ShapeDtypeStruct((n, f_out), jnp.float32),
            jax.ShapeDtypeStruct((m, f_out), jnp.float32),
        ),
        scratch_shapes=[
            pltpu.VMEM((n, f_out), jnp.float32),
            pltpu.VMEM((m, f_out), jnp.float32),
        ],
        compiler_params=pltpu.CompilerParams(
            dimension_semantics=("arbitrary",),
        ),
    )(node_input, hyperedge_input, weight,
      node_lap, node_lap, hyperedge_lap, hyperedge_lap, bias2d)
    return o1, o2
